# Initial kernel scaffold; baseline (speedup 1.0000x reference)
import jax
import jax.numpy as jnp
from jax.experimental import pallas as pl


def kernel(hidden_states, cluster_usage, embed_sum):
    raise NotImplementedError("write your pallas kernel here")


# speedup vs baseline: 1.3355x; 1.3355x over previous
import jax
import jax.numpy as jnp
from jax.experimental import pallas as pl

D = 256
K = 2048
EPS = 1e-05


def _vq_argmin_kernel(x_ref, usage_ref, esum_ref, out_ref):
    e = esum_ref[...] / jnp.clip(usage_ref[...], EPS, None)[:, None]
    e2 = jnp.sum(e * e, axis=1)[None, :]

    x = x_ref[...]
    x2 = jnp.sum(x * x, axis=1, keepdims=True)
    s = jax.lax.dot_general(x, e, (((1,), (1,)), ((), ())),
                            preferred_element_type=jnp.float32)
    d2 = (x2 + e2) - 2.0 * s
    d2 = jnp.maximum(d2, 0.0)

    m = jnp.min(d2, axis=1, keepdims=True)
    iota = jax.lax.broadcasted_iota(jnp.int32, d2.shape, 1)
    idx = jnp.min(jnp.where(d2 == m, iota, K), axis=1)
    out_ref[...] = idx[None, None, :].astype(jnp.int32)


def kernel(hidden_states, cluster_usage, embed_sum):
    shape = hidden_states.shape
    rows = shape[0] * shape[1]
    R = 1024
    nt = rows // R

    x = hidden_states.reshape(rows, D)
    out = pl.pallas_call(
        _vq_argmin_kernel,
        grid=(nt,),
        in_specs=[
            pl.BlockSpec((R, D), lambda i: (i, 0)),
            pl.BlockSpec((K,), lambda i: (0,)),
            pl.BlockSpec((K, D), lambda i: (0, 0)),
        ],
        out_specs=pl.BlockSpec((1, 1, R), lambda i: (i, 0, 0)),
        out_shape=jax.ShapeDtypeStruct((nt, 1, R), jnp.int32),
    )(x, cluster_usage, embed_sum)
    return out.reshape(shape[:-1])


# speedup vs baseline: 2.6427x; 1.9788x over previous
import jax
import jax.numpy as jnp
from jax.experimental import pallas as pl

D = 256
K = 2048
EPS = 1e-05
C = 8


def _prep_kernel(usage_ref, esum_ref, nege_ref, he2_ref):
    e = esum_ref[...] / jnp.clip(usage_ref[...], EPS, None)[:, None]
    nege_ref[...] = -e
    he2_ref[...] = 0.5 * jnp.sum(e * e, axis=1, keepdims=True)


def _argmin_kernel(x_ref, nege_ref, he2_ref, out_ref):
    s = jax.lax.dot_general(nege_ref[...], x_ref[...],
                            (((1,), (1,)), ((), ())),
                            preferred_element_type=jnp.float32)
    he2 = he2_ref[...]

    best_v = he2[0:C] + s[0:C]
    best_c = jnp.zeros(best_v.shape, jnp.int32)
    for c in range(1, K // C):
        v = he2[c * C:(c + 1) * C] + s[c * C:(c + 1) * C]
        take = v < best_v
        best_v = jnp.minimum(best_v, v)
        best_c = jnp.where(take, c, best_c)

    m = jnp.min(best_v, axis=0, keepdims=True)
    sub = jax.lax.broadcasted_iota(jnp.int32, best_c.shape, 0)
    cand = best_c * C + sub
    idx = jnp.min(jnp.where(best_v == m, cand, K), axis=0)
    out_ref[...] = idx[None, None, :].astype(jnp.int32)


def kernel(hidden_states, cluster_usage, embed_sum):
    shape = hidden_states.shape
    rows = shape[0] * shape[1]
    R = 1024
    nt = rows // R

    neg_e, half_e2 = pl.pallas_call(
        _prep_kernel,
        in_specs=[
            pl.BlockSpec((K,), lambda: (0,)),
            pl.BlockSpec((K, D), lambda: (0, 0)),
        ],
        out_specs=[
            pl.BlockSpec((K, D), lambda: (0, 0)),
            pl.BlockSpec((K, 1), lambda: (0, 0)),
        ],
        out_shape=[
            jax.ShapeDtypeStruct((K, D), jnp.float32),
            jax.ShapeDtypeStruct((K, 1), jnp.float32),
        ],
    )(cluster_usage, embed_sum)

    x = hidden_states.reshape(rows, D)
    out = pl.pallas_call(
        _argmin_kernel,
        grid=(nt,),
        in_specs=[
            pl.BlockSpec((R, D), lambda i: (i, 0)),
            pl.BlockSpec((K, D), lambda i: (0, 0)),
            pl.BlockSpec((K, 1), lambda i: (0, 0)),
        ],
        out_specs=pl.BlockSpec((1, 1, R), lambda i: (i, 0, 0)),
        out_shape=jax.ShapeDtypeStruct((nt, 1, R), jnp.int32),
    )(x, neg_e, half_e2)
    return out.reshape(shape[:-1])


# speedup vs baseline: 2.6768x; 1.0129x over previous
import jax
import jax.numpy as jnp
from jax.experimental import pallas as pl
from jax.experimental.pallas import tpu as pltpu

D = 256
K = 2048
EPS = 1e-05
C = 8


def _prep_kernel(usage_ref, esum_ref, nege_ref, he2_ref):
    e = esum_ref[...] / jnp.clip(usage_ref[...], EPS, None)[:, None]
    nege_ref[...] = -e
    he2_ref[...] = 0.5 * jnp.sum(e * e, axis=1, keepdims=True)


def _argmin_kernel(x_ref, nege_ref, he2_ref, out_ref):
    s = jax.lax.dot_general(nege_ref[...], x_ref[...],
                            (((1,), (1,)), ((), ())),
                            preferred_element_type=jnp.float32)
    he2 = he2_ref[...]

    best_v = he2[0:C] + s[0:C]
    best_c = jnp.zeros(best_v.shape, jnp.int32)
    for c in range(1, K // C):
        v = he2[c * C:(c + 1) * C] + s[c * C:(c + 1) * C]
        take = v < best_v
        best_v = jnp.minimum(best_v, v)
        best_c = jnp.where(take, c, best_c)

    m = jnp.min(best_v, axis=0, keepdims=True)
    sub = jax.lax.broadcasted_iota(jnp.int32, best_c.shape, 0)
    cand = best_c * C + sub
    idx = jnp.min(jnp.where(best_v == m, cand, K), axis=0)
    out_ref[...] = idx[None, None, :].astype(jnp.int32)


def kernel(hidden_states, cluster_usage, embed_sum):
    shape = hidden_states.shape
    rows = shape[0] * shape[1]
    R = 1024
    nt = rows // R

    neg_e, half_e2 = pl.pallas_call(
        _prep_kernel,
        in_specs=[
            pl.BlockSpec((K,), lambda: (0,)),
            pl.BlockSpec((K, D), lambda: (0, 0)),
        ],
        out_specs=[
            pl.BlockSpec((K, D), lambda: (0, 0)),
            pl.BlockSpec((K, 1), lambda: (0, 0)),
        ],
        out_shape=[
            jax.ShapeDtypeStruct((K, D), jnp.float32),
            jax.ShapeDtypeStruct((K, 1), jnp.float32),
        ],
    )(cluster_usage, embed_sum)

    x = hidden_states.reshape(rows, D)
    out = pl.pallas_call(
        _argmin_kernel,
        grid=(nt,),
        in_specs=[
            pl.BlockSpec((R, D), lambda i: (i, 0)),
            pl.BlockSpec((K, D), lambda i: (0, 0)),
            pl.BlockSpec((K, 1), lambda i: (0, 0)),
        ],
        out_specs=pl.BlockSpec((1, 1, R), lambda i: (i, 0, 0)),
        out_shape=jax.ShapeDtypeStruct((nt, 1, R), jnp.int32),
        compiler_params=pltpu.CompilerParams(
            dimension_semantics=("parallel",)),
    )(x, neg_e, half_e2)
    return out.reshape(shape[:-1])


# speedup vs baseline: 3.1694x; 1.1840x over previous
import jax
import jax.numpy as jnp
from jax.experimental import pallas as pl
from jax.experimental.pallas import tpu as pltpu

D = 256
K = 2048
EPS = 1e-05
C = 8


def _encode_kernel(usage_ref, esum_ref, x_ref, out_ref, nege_ref, he2_ref):
    @pl.when(pl.program_id(0) == 0)
    def _prep():
        e = esum_ref[...] / jnp.clip(usage_ref[...], EPS, None)[:, None]
        nege_ref[...] = -e
        he2_ref[...] = 0.5 * jnp.sum(e * e, axis=1, keepdims=True)

    s = jax.lax.dot_general(nege_ref[...], x_ref[...],
                            (((1,), (1,)), ((), ())),
                            preferred_element_type=jnp.float32)
    he2 = he2_ref[...]

    best_v = he2[0:C] + s[0:C]
    best_c = jnp.zeros(best_v.shape, jnp.int32)
    for c in range(1, K // C):
        v = he2[c * C:(c + 1) * C] + s[c * C:(c + 1) * C]
        take = v < best_v
        best_v = jnp.minimum(best_v, v)
        best_c = jnp.where(take, c, best_c)

    m = jnp.min(best_v, axis=0, keepdims=True)
    sub = jax.lax.broadcasted_iota(jnp.int32, best_c.shape, 0)
    cand = best_c * C + sub
    idx = jnp.min(jnp.where(best_v == m, cand, K), axis=0)
    out_ref[...] = idx[None, None, :].astype(jnp.int32)


def kernel(hidden_states, cluster_usage, embed_sum):
    shape = hidden_states.shape
    rows = shape[0] * shape[1]
    R = 1024
    nt = rows // R

    x = hidden_states.reshape(rows, D)
    out = pl.pallas_call(
        _encode_kernel,
        grid=(nt,),
        in_specs=[
            pl.BlockSpec((K,), lambda i: (0,)),
            pl.BlockSpec((K, D), lambda i: (0, 0)),
            pl.BlockSpec((R, D), lambda i: (i, 0)),
        ],
        out_specs=pl.BlockSpec((1, 1, R), lambda i: (i, 0, 0)),
        out_shape=jax.ShapeDtypeStruct((nt, 1, R), jnp.int32),
        scratch_shapes=[
            pltpu.VMEM((K, D), jnp.float32),
            pltpu.VMEM((K, 1), jnp.float32),
        ],
        compiler_params=pltpu.CompilerParams(
            dimension_semantics=("arbitrary",)),
    )(cluster_usage, embed_sum, x)
    return out.reshape(shape[:-1])


# speedup vs baseline: 3.3534x; 1.0581x over previous
import jax
import jax.numpy as jnp
from jax.experimental import pallas as pl
from jax.experimental.pallas import tpu as pltpu

D = 256
K = 2048
EPS = 1e-05
C = 8


def _encode_kernel(usage_ref, esum_ref, x_ref, out_ref, nege_ref, he2_ref):
    @pl.when(pl.program_id(0) == 0)
    def _prep():
        e = esum_ref[...] / jnp.clip(usage_ref[...], EPS, None)[:, None]
        nege_ref[...] = -e
        he2_ref[...] = 0.5 * jnp.sum(e * e, axis=1, keepdims=True)

    s = jax.lax.dot_general(nege_ref[...], x_ref[...],
                            (((1,), (1,)), ((), ())),
                            preferred_element_type=jnp.float32)
    he2 = he2_ref[...]

    best_v = he2[0:C] + s[0:C]
    best_c = jnp.zeros(best_v.shape, jnp.int32)
    for c in range(1, K // C):
        v = he2[c * C:(c + 1) * C] + s[c * C:(c + 1) * C]
        take = v < best_v
        best_v = jnp.minimum(best_v, v)
        best_c = jnp.where(take, c, best_c)

    m = jnp.min(best_v, axis=0, keepdims=True)
    sub = jax.lax.broadcasted_iota(jnp.int32, best_c.shape, 0)
    cand = best_c * C + sub
    idx = jnp.min(jnp.where(best_v == m, cand, K), axis=0)
    out_ref[...] = idx[None, None, :].astype(jnp.int32)


def kernel(hidden_states, cluster_usage, embed_sum):
    shape = hidden_states.shape
    rows = shape[0] * shape[1]
    R = 2048
    nt = rows // R

    x = hidden_states.reshape(rows, D)
    out = pl.pallas_call(
        _encode_kernel,
        grid=(nt,),
        in_specs=[
            pl.BlockSpec((K,), lambda i: (0,)),
            pl.BlockSpec((K, D), lambda i: (0, 0)),
            pl.BlockSpec((R, D), lambda i: (i, 0)),
        ],
        out_specs=pl.BlockSpec((1, 1, R), lambda i: (i, 0, 0)),
        out_shape=jax.ShapeDtypeStruct((nt, 1, R), jnp.int32),
        scratch_shapes=[
            pltpu.VMEM((K, D), jnp.float32),
            pltpu.VMEM((K, 1), jnp.float32),
        ],
        compiler_params=pltpu.CompilerParams(
            dimension_semantics=("arbitrary",)),
    )(cluster_usage, embed_sum, x)
    return out.reshape(shape[:-1])
